# baseline (speedup 1.0000x reference)
import jax
import jax.numpy as jnp
import numpy as np
from jax.experimental import pallas as pl
from jax.experimental.pallas import tpu as pltpu

_LTM_SIZE = 1000000
_EMB_DIM = 128
_BATCH = 32
_TEMPERATURE = 0.11 - float(np.log10(float(_LTM_SIZE))) * 0.01
_INV_T = 1.0 / _TEMPERATURE

_CHUNK = 20000
_NCHUNK = _LTM_SIZE // _CHUNK


def _attn_kernel(q_ref, k_ref, v_ref, o_ref, acc_ref, den_ref):
    j = pl.program_id(0)

    q = q_ref[...]
    norm = jnp.sqrt(jnp.sum(q * q, axis=1, keepdims=True))
    qs = (q / jnp.maximum(norm, 1e-12)) * _INV_T

    k = k_ref[...]
    s = jax.lax.dot_general(
        qs, k, (((1,), (1,)), ((), ())), preferred_element_type=jnp.float32
    )
    p = jnp.exp(s - _INV_T)
    pv = jax.lax.dot_general(
        p, v_ref[...], (((1,), (0,)), ((), ())), preferred_element_type=jnp.float32
    )
    psum = jnp.broadcast_to(
        jnp.sum(p, axis=1, keepdims=True), (_BATCH, _EMB_DIM)
    )

    @pl.when(j == 0)
    def _init():
        acc_ref[...] = pv
        den_ref[...] = psum

    @pl.when(j != 0)
    def _accum():
        acc_ref[...] += pv
        den_ref[...] += psum

    @pl.when(j == _NCHUNK - 1)
    def _finish():
        o_ref[...] = acc_ref[...] / den_ref[...]


def kernel(encoded_state, keys, values):
    return pl.pallas_call(
        _attn_kernel,
        grid=(_NCHUNK,),
        in_specs=[
            pl.BlockSpec((_BATCH, _EMB_DIM), lambda j: (0, 0)),
            pl.BlockSpec((_CHUNK, _EMB_DIM), lambda j: (j, 0)),
            pl.BlockSpec((_CHUNK, _EMB_DIM), lambda j: (j, 0)),
        ],
        out_specs=pl.BlockSpec((_BATCH, _EMB_DIM), lambda j: (0, 0)),
        out_shape=jax.ShapeDtypeStruct((_BATCH, _EMB_DIM), jnp.float32),
        scratch_shapes=[
            pltpu.VMEM((_BATCH, _EMB_DIM), jnp.float32),
            pltpu.VMEM((_BATCH, _EMB_DIM), jnp.float32),
        ],
        compiler_params=pltpu.CompilerParams(
            dimension_semantics=("arbitrary",),
        ),
    )(encoded_state, keys, values)


# speedup vs baseline: 1.0011x; 1.0011x over previous
import jax
import jax.numpy as jnp
import numpy as np
from jax.experimental import pallas as pl
from jax.experimental.pallas import tpu as pltpu

_LTM_SIZE = 1000000
_EMB_DIM = 128
_BATCH = 32
_TEMPERATURE = 0.11 - float(np.log10(float(_LTM_SIZE))) * 0.01
_INV_T = 1.0 / _TEMPERATURE

_CHUNK = 8000
_NCHUNK = _LTM_SIZE // _CHUNK


def _attn_kernel(q_ref, k_ref, v_ref, o_ref, acc_ref, den_ref):
    j = pl.program_id(0)

    q = q_ref[...]
    norm = jnp.sqrt(jnp.sum(q * q, axis=1, keepdims=True))
    qs = (q / jnp.maximum(norm, 1e-12)) * _INV_T

    k = k_ref[...]
    s = jax.lax.dot_general(
        qs, k, (((1,), (1,)), ((), ())), preferred_element_type=jnp.float32
    )
    p = jnp.exp(s - _INV_T)
    pv = jax.lax.dot_general(
        p, v_ref[...], (((1,), (0,)), ((), ())), preferred_element_type=jnp.float32
    )
    psum = jnp.broadcast_to(
        jnp.sum(p, axis=1, keepdims=True), (_BATCH, _EMB_DIM)
    )

    @pl.when(j == 0)
    def _init():
        acc_ref[...] = pv
        den_ref[...] = psum

    @pl.when(j != 0)
    def _accum():
        acc_ref[...] += pv
        den_ref[...] += psum

    @pl.when(j == _NCHUNK - 1)
    def _finish():
        o_ref[...] = acc_ref[...] / den_ref[...]


def kernel(encoded_state, keys, values):
    return pl.pallas_call(
        _attn_kernel,
        grid=(_NCHUNK,),
        in_specs=[
            pl.BlockSpec((_BATCH, _EMB_DIM), lambda j: (0, 0)),
            pl.BlockSpec((_CHUNK, _EMB_DIM), lambda j: (j, 0)),
            pl.BlockSpec((_CHUNK, _EMB_DIM), lambda j: (j, 0)),
        ],
        out_specs=pl.BlockSpec((_BATCH, _EMB_DIM), lambda j: (0, 0)),
        out_shape=jax.ShapeDtypeStruct((_BATCH, _EMB_DIM), jnp.float32),
        scratch_shapes=[
            pltpu.VMEM((_BATCH, _EMB_DIM), jnp.float32),
            pltpu.VMEM((_BATCH, _EMB_DIM), jnp.float32),
        ],
        compiler_params=pltpu.CompilerParams(
            dimension_semantics=("arbitrary",),
        ),
    )(encoded_state, keys, values)


# speedup vs baseline: 1.0160x; 1.0148x over previous
import jax
import jax.numpy as jnp
import numpy as np
from jax.experimental import pallas as pl
from jax.experimental.pallas import tpu as pltpu

_LTM_SIZE = 1000000
_EMB_DIM = 128
_BATCH = 32
_TEMPERATURE = 0.11 - float(np.log10(float(_LTM_SIZE))) * 0.01
_INV_T = 1.0 / _TEMPERATURE

_CHUNK = 10000
_NCHUNK = _LTM_SIZE // _CHUNK


def _attn_kernel(q_ref, k_ref, v_ref, o_ref, acc_ref, den_ref):
    j = pl.program_id(0)

    q = q_ref[...]
    norm = jnp.sqrt(jnp.sum(q * q, axis=1, keepdims=True))
    qs = (q / jnp.maximum(norm, 1e-12)) * _INV_T

    k = k_ref[...]
    s = jax.lax.dot_general(
        qs, k, (((1,), (1,)), ((), ())), preferred_element_type=jnp.float32
    )
    p = jnp.exp(s - _INV_T)
    pv = jax.lax.dot_general(
        p, v_ref[...], (((1,), (0,)), ((), ())), preferred_element_type=jnp.float32
    )
    psum = jnp.broadcast_to(
        jnp.sum(p, axis=1, keepdims=True), (_BATCH, _EMB_DIM)
    )

    @pl.when(j == 0)
    def _init():
        acc_ref[...] = pv
        den_ref[...] = psum

    @pl.when(j != 0)
    def _accum():
        acc_ref[...] += pv
        den_ref[...] += psum

    @pl.when(j == _NCHUNK - 1)
    def _finish():
        o_ref[...] = acc_ref[...] / den_ref[...]


def kernel(encoded_state, keys, values):
    return pl.pallas_call(
        _attn_kernel,
        grid=(_NCHUNK,),
        in_specs=[
            pl.BlockSpec((_BATCH, _EMB_DIM), lambda j: (0, 0)),
            pl.BlockSpec((_CHUNK, _EMB_DIM), lambda j: (j, 0)),
            pl.BlockSpec((_CHUNK, _EMB_DIM), lambda j: (j, 0)),
        ],
        out_specs=pl.BlockSpec((_BATCH, _EMB_DIM), lambda j: (0, 0)),
        out_shape=jax.ShapeDtypeStruct((_BATCH, _EMB_DIM), jnp.float32),
        scratch_shapes=[
            pltpu.VMEM((_BATCH, _EMB_DIM), jnp.float32),
            pltpu.VMEM((_BATCH, _EMB_DIM), jnp.float32),
        ],
        compiler_params=pltpu.CompilerParams(
            dimension_semantics=("arbitrary",),
        ),
    )(encoded_state, keys, values)


# speedup vs baseline: 1.0325x; 1.0162x over previous
import jax
import jax.numpy as jnp
import numpy as np
from jax.experimental import pallas as pl
from jax.experimental.pallas import tpu as pltpu

_LTM_SIZE = 1000000
_EMB_DIM = 128
_BATCH = 32
_TEMPERATURE = 0.11 - float(np.log10(float(_LTM_SIZE))) * 0.01
_INV_T = 1.0 / _TEMPERATURE

_CHUNK = 10000
_NCHUNK = _LTM_SIZE // _CHUNK


def _attn_kernel(q_ref, ka_ref, kb_ref, va_ref, vb_ref, o_ref, acc_ref, den_ref):
    j = pl.program_id(0)

    q = q_ref[...]
    norm = jnp.sqrt(jnp.sum(q * q, axis=1, keepdims=True))
    qs = (q / jnp.maximum(norm, 1e-12)) * _INV_T

    pv = jnp.zeros((_BATCH, _EMB_DIM), jnp.float32)
    ps = jnp.zeros((_BATCH, 1), jnp.float32)
    for k_ref, v_ref in ((ka_ref, va_ref), (kb_ref, vb_ref)):
        s = jax.lax.dot_general(
            qs, k_ref[...], (((1,), (1,)), ((), ())),
            preferred_element_type=jnp.float32,
        )
        p = jnp.exp(s - _INV_T)
        pv = pv + jax.lax.dot_general(
            p, v_ref[...], (((1,), (0,)), ((), ())),
            preferred_element_type=jnp.float32,
        )
        ps = ps + jnp.sum(p, axis=1, keepdims=True)
    psum = jnp.broadcast_to(ps, (_BATCH, _EMB_DIM))

    @pl.when(j == 0)
    def _init():
        acc_ref[...] = pv
        den_ref[...] = psum

    @pl.when(j != 0)
    def _accum():
        acc_ref[...] += pv
        den_ref[...] += psum

    @pl.when(j == _NCHUNK - 1)
    def _finish():
        o_ref[...] = acc_ref[...] / den_ref[...]


def kernel(encoded_state, keys, values):
    return pl.pallas_call(
        _attn_kernel,
        grid=(_NCHUNK,),
        in_specs=[
            pl.BlockSpec((_BATCH, _EMB_DIM), lambda j: (0, 0)),
            pl.BlockSpec((_CHUNK // 2, _EMB_DIM), lambda j: (2 * j, 0)),
            pl.BlockSpec((_CHUNK // 2, _EMB_DIM), lambda j: (2 * j + 1, 0)),
            pl.BlockSpec((_CHUNK // 2, _EMB_DIM), lambda j: (2 * j, 0)),
            pl.BlockSpec((_CHUNK // 2, _EMB_DIM), lambda j: (2 * j + 1, 0)),
        ],
        out_specs=pl.BlockSpec((_BATCH, _EMB_DIM), lambda j: (0, 0)),
        out_shape=jax.ShapeDtypeStruct((_BATCH, _EMB_DIM), jnp.float32),
        scratch_shapes=[
            pltpu.VMEM((_BATCH, _EMB_DIM), jnp.float32),
            pltpu.VMEM((_BATCH, _EMB_DIM), jnp.float32),
        ],
        compiler_params=pltpu.CompilerParams(
            dimension_semantics=("arbitrary",),
        ),
    )(encoded_state, keys, keys, values, values)
